# baseline (speedup 1.0000x reference)
import functools

import jax
import jax.numpy as jnp
from jax import lax
from jax.experimental import pallas as pl
from jax.experimental.pallas import tpu as pltpu
from jax.experimental.pallas import tpu_sc as plsc

D = 32
NC = 2
NS = 16
NW = NC * NS
G = 128
NB = 4


@functools.partial(jax.jit, static_argnames=("n_s", "n_b"))
def _gather_t(idx, table, n_s, n_b):
    bt_per_s = n_b // G
    n_blocks = n_s * bt_per_s
    blk_per_w = n_blocks // NW
    idx_per_w = blk_per_w * G
    n_groups = blk_per_w // NB
    mesh = plsc.VectorSubcoreMesh(core_axis_name="c", subcore_axis_name="s")

    @functools.partial(
        pl.kernel,
        mesh=mesh,
        out_type=jax.ShapeDtypeStruct((n_s, D // 8, bt_per_s, 8, G), jnp.float32),
        scratch_types=[
            pltpu.VMEM((idx_per_w,), jnp.int32),
            pltpu.VMEM((NB, G, D), jnp.float32),
            pltpu.VMEM((NB, D // 8, 8, G), jnp.float32),
            [pltpu.SemaphoreType.DMA] * NB,
            [pltpu.SemaphoreType.DMA] * NB,
        ],
        compiler_params=pltpu.CompilerParams(
            use_tc_tiling_on_sc=False, needs_layout_passes=False
        ),
    )
    def emb(idx_hbm, table_hbm, out_hbm, idx_v, rows_v, stg_v, gsems, ssems):
        wid = lax.axis_index("s") * NC + lax.axis_index("c")
        base_blk = wid * blk_per_w
        iota = lax.iota(jnp.int32, 16)

        pltpu.sync_copy(idx_hbm.at[pl.ds(wid * idx_per_w, idx_per_w)], idx_v)

        def fire(local_blk, slot):
            pltpu.async_copy(
                table_hbm.at[idx_v.at[pl.ds(local_blk * G, G)]],
                rows_v.at[slot],
                gsems[slot],
            )

        for u in range(NB):
            fire(u, u)

        def body(g, carry):
            for u in range(NB):
                lb = g * NB + u
                blk = base_blk + lb
                s = blk // bt_per_s
                bt = blk % bt_per_s
                pltpu.make_async_copy(
                    table_hbm.at[idx_v.at[pl.ds(0, G)]], rows_v.at[u], gsems[u]
                ).wait()

                @pl.when(g > 0)
                def _():
                    for dt in range(D // 8):
                        pltpu.make_async_copy(
                            stg_v.at[u, dt], out_hbm.at[0, dt, 0], ssems[u]
                        ).wait()

                for dt in range(D // 8):
                    for di in range(8):
                        d = dt * 8 + di
                        for l in range(G // 16):
                            v = plsc.load_gather(
                                rows_v.at[u],
                                [iota + (l * 16), jnp.full((16,), d, jnp.int32)],
                            )
                            stg_v[u, dt, di, pl.ds(l * 16, 16)] = v
                for dt in range(D // 8):
                    pltpu.async_copy(
                        stg_v.at[u, dt], out_hbm.at[s, dt, bt], ssems[u]
                    )

                @pl.when(lb + NB < blk_per_w)
                def _():
                    fire(lb + NB, u)
            return carry

        lax.fori_loop(0, n_groups, body, 0)

        for u in range(NB):
            for dt in range(D // 8):
                pltpu.make_async_copy(
                    stg_v.at[u, dt], out_hbm.at[0, dt, 0], ssems[u]
                ).wait()

    return emb(idx, table)


def kernel(x, table):
    b, s = x.shape
    idx = jnp.reshape(jnp.transpose(x), (b * s,)).astype(jnp.int32)
    out5 = _gather_t(idx, table, s, b)
    return jnp.reshape(jnp.transpose(out5, (2, 4, 0, 1, 3)), (b, s, D))


# speedup vs baseline: 1.2123x; 1.2123x over previous
import functools

import jax
import jax.numpy as jnp
from jax import lax
from jax.experimental import pallas as pl
from jax.experimental.pallas import tpu as pltpu
from jax.experimental.pallas import tpu_sc as plsc

D = 32
NC = 2
NS = 16
NW = NC * NS
G = 128
NB = 5


@functools.partial(jax.jit, static_argnames=("n_s", "n_b"))
def _gather_t(idx, table, n_s, n_b):
    bt_per_s = n_b // G
    n_blocks = n_s * bt_per_s
    blk_per_w = n_blocks // NW
    idx_per_w = blk_per_w * G
    n_groups = blk_per_w // NB
    mesh = plsc.VectorSubcoreMesh(core_axis_name="c", subcore_axis_name="s")

    @functools.partial(
        pl.kernel,
        mesh=mesh,
        out_type=jax.ShapeDtypeStruct((n_s, D // 8, bt_per_s, 8, G), jnp.float32),
        scratch_types=[
            pltpu.VMEM((idx_per_w,), jnp.int32),
            pltpu.VMEM((NB, G, D), jnp.float32),
            pltpu.VMEM((NB, D // 8, 8, G), jnp.float32),
            [pltpu.SemaphoreType.DMA] * NB,
            [pltpu.SemaphoreType.DMA] * NB,
        ],
        compiler_params=pltpu.CompilerParams(
            use_tc_tiling_on_sc=False,
            needs_layout_passes=False,
            disable_bounds_checks=True,
        ),
    )
    def emb(idx_hbm, table_hbm, out_hbm, idx_v, rows_v, stg_v, gsems, ssems):
        wid = lax.axis_index("s") * NC + lax.axis_index("c")
        base_blk = wid * blk_per_w
        bases = [lax.iota(jnp.int32, 16) + l * 16 for l in range(G // 16)]

        pltpu.sync_copy(idx_hbm.at[pl.ds(wid * idx_per_w, idx_per_w)], idx_v)

        def fire(local_blk, slot):
            pltpu.async_copy(
                table_hbm.at[idx_v.at[pl.ds(local_blk * G, G)]],
                rows_v.at[slot],
                gsems[slot],
            )

        for u in range(NB):
            fire(u, u)

        def body(g, carry):
            for u in range(NB):
                b = g * NB + u
                blk = base_blk + b
                s = blk // bt_per_s
                bt = blk % bt_per_s
                pltpu.make_async_copy(
                    table_hbm.at[idx_v.at[pl.ds(0, G)]], rows_v.at[u], gsems[u]
                ).wait()

                @pl.when(g > 0)
                def _():
                    pltpu.make_async_copy(
                        stg_v.at[0], out_hbm.at[0, :, 0], ssems[u]
                    ).wait()

                for dd in range(D):
                    col = jnp.full((16,), dd, jnp.int32)
                    vs = [
                        plsc.load_gather(rows_v.at[u], [bases[l], col])
                        for l in range(G // 16)
                    ]
                    for l, v in enumerate(vs):
                        stg_v[u, dd // 8, dd % 8, pl.ds(l * 16, 16)] = v

                pltpu.async_copy(stg_v.at[u], out_hbm.at[s, :, bt], ssems[u])

                @pl.when(b + NB < blk_per_w)
                def _():
                    fire(b + NB, u)
            return carry

        lax.fori_loop(0, n_groups, body, 0)

        for u in range(NB):
            pltpu.make_async_copy(
                stg_v.at[0], out_hbm.at[0, :, 0], ssems[u]
            ).wait()

    return emb(idx, table)


def kernel(x, table):
    b, s = x.shape
    idx = jnp.reshape(jnp.transpose(x), (b * s,)).astype(jnp.int32)
    out5 = _gather_t(idx, table, s, b)
    return jnp.reshape(jnp.transpose(out5, (2, 4, 0, 1, 3)), (b, s, D))
